# baseline (speedup 1.0000x reference)
import functools

import jax
import jax.numpy as jnp
from jax import lax
from jax.experimental import pallas as pl
from jax.experimental.pallas import tpu as pltpu
from jax.experimental.pallas import tpu_sc as plsc

D_PROJ = 128
EMB_SCALE = float(D_PROJ) ** 0.5

_C2_BASE, _C2_CHUNK, _C2_G = 100000, 25000, 16
_C3_BASE, _C3_CHUNK, _C3_G = 500224, 15632, 32
_P_ROWS = _C3_BASE + _C3_G * _C3_CHUNK

_NC, _NS = 2, 16
_NW = _NC * _NS
_N_TOK = 819200
_IDX_ROWS = _N_TOK // 128
_RPW = _IDX_ROWS // _NW


def _p01_body(e0, e1, p0, p1, out):
    g = pl.program_id(0)

    def mm(e, p):
        out[...] = lax.dot_general(
            e[...], p[...], (((1,), (1,)), ((), ())),
            preferred_element_type=jnp.float32)

    @pl.when(g < 1)
    def _():
        mm(e0, p0)

    @pl.when(g >= 1)
    def _():
        mm(e1, p1)


def _project01(e0, e1, p0s, p1s):
    return pl.pallas_call(
        _p01_body,
        grid=(5,),
        in_specs=[
            pl.BlockSpec((20000, 128), lambda g: (0, 0)),
            pl.BlockSpec((20000, 32), lambda g: (jnp.clip(g - 1, 0, 3), 0)),
            pl.BlockSpec((128, 128), lambda g: (0, 0)),
            pl.BlockSpec((128, 32), lambda g: (0, 0)),
        ],
        out_specs=pl.BlockSpec((20000, 128), lambda g: (g, 0)),
        out_shape=jax.ShapeDtypeStruct((_P_ROWS, 128), jnp.float32),
    )(e0, e1, p0s, p1s)


def _packed_body(p_in, ec, w, out):
    del p_in
    out[...] = lax.dot_general(
        ec[...], w[...], (((1,), (0,)), ((), ())),
        preferred_element_type=jnp.float32)


def _project_packed(P, ec, wstack, chunk, base_blk, n_chunks):
    return pl.pallas_call(
        _packed_body,
        grid=(n_chunks,),
        in_specs=[
            pl.BlockSpec(memory_space=pl.ANY),
            pl.BlockSpec(ec.shape, lambda g: (0, 0)),
            pl.BlockSpec((128, 128), lambda g: (g, 0)),
        ],
        out_specs=pl.BlockSpec((chunk, 128), lambda g: (base_blk + g, 0)),
        out_shape=jax.ShapeDtypeStruct((_P_ROWS, 128), jnp.float32),
        input_output_aliases={0: 0},
    )(P, ec, wstack)


def _pi_body(v_ref, out_ref):
    v = v_ref[...]
    r2 = v - _C2_BASE
    r3 = v - 500000
    pi2 = _C2_BASE + (r2 & (_C2_G - 1)) * _C2_CHUNK + (r2 >> 4)
    pi3 = _C3_BASE + (r3 & (_C3_G - 1)) * _C3_CHUNK + (r3 >> 5)
    out_ref[...] = jnp.where(
        v < _C2_BASE, v, jnp.where(v < 500000, pi2, pi3))


def _pi(idx):
    return pl.pallas_call(
        _pi_body,
        out_shape=jax.ShapeDtypeStruct((_IDX_ROWS, 128), jnp.int32),
    )(idx)


_NBUF = 4


def _gather(P, idx):
    mesh = plsc.VectorSubcoreMesh(core_axis_name="c", subcore_axis_name="s")

    @functools.partial(
        pl.kernel,
        out_type=jax.ShapeDtypeStruct((_N_TOK, 128), jnp.float32),
        mesh=mesh,
        scratch_types=[
            pltpu.VMEM((_RPW, 128), jnp.int32),
            pltpu.VMEM((_NBUF, 128, 128), jnp.float32),
            [pltpu.SemaphoreType.DMA] * _NBUF,
            [pltpu.SemaphoreType.DMA] * _NBUF,
        ],
    )
    def gk(p_hbm, idx_hbm, out_hbm, idx_v, rows_v, gsems, wsems):
        wid = lax.axis_index("s") * _NC + lax.axis_index("c")
        row0 = wid * _RPW
        pltpu.sync_copy(idx_hbm.at[pl.ds(row0, _RPW)], idx_v)

        def gs(c, b):
            pltpu.make_async_copy(
                p_hbm.at[idx_v.at[c]], rows_v.at[b], gsems[b]).start()

        def gw(b):
            pltpu.make_async_copy(
                p_hbm.at[idx_v.at[0]], rows_v.at[b], gsems[b]).wait()

        def ws(c, b):
            pltpu.make_async_copy(
                rows_v.at[b], out_hbm.at[pl.ds((row0 + c) * 128, 128)],
                wsems[b]).start()

        def ww(b):
            pltpu.make_async_copy(
                rows_v.at[b], out_hbm.at[pl.ds(row0 * 128, 128)],
                wsems[b]).wait()

        for b in range(_NBUF):
            gs(b, b)

        def step(k, carry):
            base = k * _NBUF
            for b in range(_NBUF):
                gw(b)
                ws(base + b, b)
            for b in range(_NBUF):
                ww(b)
                gs(base + _NBUF + b, b)
            return carry

        n_steady = _RPW // _NBUF - 1
        lax.fori_loop(0, n_steady, step, 0)

        tail = _RPW - _NBUF
        for b in range(_NBUF):
            gw(b)
            ws(tail + b, b)
        for b in range(_NBUF):
            ww(b)

    return gk(P, idx)


def _band_w(proj, d, g):
    l = jnp.arange(128)
    j = jnp.arange(g)
    k = jnp.arange(d)
    onehot = (l[None, :, None] == (d * j[:, None, None] + k[None, None, :]))
    w = jnp.einsum("jlk,ko->jlo", onehot.astype(jnp.float32),
                   proj.T.astype(jnp.float32) * EMB_SCALE)
    return w.reshape(g * 128, 128)


def kernel(inp, emb_0, emb_1, emb_2, emb_3, proj_0, proj_1, proj_2, proj_3):
    e2c = emb_2.reshape(_C2_CHUNK, 128)
    e3c = jnp.pad(emb_3, ((0, 224), (0, 2))).reshape(_C3_CHUNK, 128)
    w2 = _band_w(proj_2, 8, _C2_G)
    w3 = _band_w(jnp.pad(proj_3, ((0, 0), (0, 2))), 4, _C3_G)
    p0s = proj_0 * EMB_SCALE
    p1s = proj_1 * EMB_SCALE

    P = _project01(emb_0, emb_1, p0s, p1s)
    P = _project_packed(P, e2c, w2, _C2_CHUNK, _C2_BASE // _C2_CHUNK, _C2_G)
    P = _project_packed(P, e3c, w3, _C3_CHUNK, _C3_BASE // _C3_CHUNK, _C3_G)

    idx = _pi(inp.astype(jnp.int32).reshape(_IDX_ROWS, 128))
    out = _gather(P, idx)
    return out.reshape(inp.shape + (D_PROJ,))


# speedup vs baseline: 1.0062x; 1.0062x over previous
import functools

import jax
import jax.numpy as jnp
from jax import lax
from jax.experimental import pallas as pl
from jax.experimental.pallas import tpu as pltpu
from jax.experimental.pallas import tpu_sc as plsc

D_PROJ = 128
EMB_SCALE = float(D_PROJ) ** 0.5

_C2_BASE, _C2_CHUNK, _C2_G = 100000, 25000, 16
_C3_BASE, _C3_CHUNK, _C3_G = 500224, 7816, 64
_P_ROWS = _C3_BASE + _C3_G * _C3_CHUNK

_NC, _NS = 2, 16
_NW = _NC * _NS
_N_TOK = 819200
_IDX_ROWS = _N_TOK // 128
_RPW = _IDX_ROWS // _NW


def _p01_body(e0, e1, p0, p1, out):
    g = pl.program_id(0)

    def mm(e, p):
        out[...] = lax.dot_general(
            e[...], p[...], (((1,), (1,)), ((), ())),
            preferred_element_type=jnp.float32)

    @pl.when(g < 1)
    def _():
        mm(e0, p0)

    @pl.when(g >= 1)
    def _():
        mm(e1, p1)


def _project01(e0, e1, p0s, p1s):
    return pl.pallas_call(
        _p01_body,
        grid=(5,),
        in_specs=[
            pl.BlockSpec((20000, 128), lambda g: (0, 0)),
            pl.BlockSpec((20000, 32), lambda g: (jnp.clip(g - 1, 0, 3), 0)),
            pl.BlockSpec((128, 128), lambda g: (0, 0)),
            pl.BlockSpec((128, 32), lambda g: (0, 0)),
        ],
        out_specs=pl.BlockSpec((20000, 128), lambda g: (g, 0)),
        out_shape=jax.ShapeDtypeStruct((_P_ROWS, 128), jnp.float32),
    )(e0, e1, p0s, p1s)


def _packed_body(p_in, ec, w, out):
    del p_in
    out[...] = lax.dot_general(
        ec[...], w[...], (((1,), (0,)), ((), ())),
        preferred_element_type=jnp.float32)


def _project_packed(P, ec, wstack, chunk, base_blk, n_chunks):
    return pl.pallas_call(
        _packed_body,
        grid=(n_chunks,),
        in_specs=[
            pl.BlockSpec(memory_space=pl.ANY),
            pl.BlockSpec(ec.shape, lambda g: (0, 0)),
            pl.BlockSpec((128, 128), lambda g: (g, 0)),
        ],
        out_specs=pl.BlockSpec((chunk, 128), lambda g: (base_blk + g, 0)),
        out_shape=jax.ShapeDtypeStruct((_P_ROWS, 128), jnp.float32),
        input_output_aliases={0: 0},
    )(P, ec, wstack)


def _pi_body(v_ref, out_ref):
    v = v_ref[...]
    r2 = v - _C2_BASE
    r3 = v - 500000
    pi2 = _C2_BASE + (r2 & (_C2_G - 1)) * _C2_CHUNK + (r2 >> 4)
    pi3 = _C3_BASE + (r3 & (_C3_G - 1)) * _C3_CHUNK + (r3 >> 6)
    out_ref[...] = jnp.where(
        v < _C2_BASE, v, jnp.where(v < 500000, pi2, pi3))


def _pi(idx):
    return pl.pallas_call(
        _pi_body,
        out_shape=jax.ShapeDtypeStruct((_IDX_ROWS, 128), jnp.int32),
    )(idx)


_NBUF = 4


def _gather(P, idx):
    mesh = plsc.VectorSubcoreMesh(core_axis_name="c", subcore_axis_name="s")

    @functools.partial(
        pl.kernel,
        out_type=jax.ShapeDtypeStruct((_N_TOK, 128), jnp.float32),
        mesh=mesh,
        scratch_types=[
            pltpu.VMEM((_RPW, 128), jnp.int32),
            pltpu.VMEM((_NBUF, 128, 128), jnp.float32),
            [pltpu.SemaphoreType.DMA] * _NBUF,
            [pltpu.SemaphoreType.DMA] * _NBUF,
        ],
    )
    def gk(p_hbm, idx_hbm, out_hbm, idx_v, rows_v, gsems, wsems):
        wid = lax.axis_index("s") * _NC + lax.axis_index("c")
        row0 = wid * _RPW
        pltpu.sync_copy(idx_hbm.at[pl.ds(row0, _RPW)], idx_v)

        def gs(c, b):
            pltpu.make_async_copy(
                p_hbm.at[idx_v.at[c]], rows_v.at[b], gsems[b]).start()

        def gw(b):
            pltpu.make_async_copy(
                p_hbm.at[idx_v.at[0]], rows_v.at[b], gsems[b]).wait()

        def ws(c, b):
            pltpu.make_async_copy(
                rows_v.at[b], out_hbm.at[pl.ds((row0 + c) * 128, 128)],
                wsems[b]).start()

        def ww(b):
            pltpu.make_async_copy(
                rows_v.at[b], out_hbm.at[pl.ds(row0 * 128, 128)],
                wsems[b]).wait()

        for b in range(_NBUF):
            gs(b, b)

        def step(k, carry):
            base = k * _NBUF
            for b in range(_NBUF):
                gw(b)
                ws(base + b, b)
            for b in range(_NBUF):
                ww(b)
                gs(base + _NBUF + b, b)
            return carry

        n_steady = _RPW // _NBUF - 1
        lax.fori_loop(0, n_steady, step, 0)

        tail = _RPW - _NBUF
        for b in range(_NBUF):
            gw(b)
            ws(tail + b, b)
        for b in range(_NBUF):
            ww(b)

    return gk(P, idx)


def _band_w(proj, d, g):
    l = jnp.arange(128)
    j = jnp.arange(g)
    k = jnp.arange(d)
    onehot = (l[None, :, None] == (d * j[:, None, None] + k[None, None, :]))
    w = jnp.einsum("jlk,ko->jlo", onehot.astype(jnp.float32),
                   proj.T.astype(jnp.float32) * EMB_SCALE)
    return w.reshape(g * 128, 128)


def kernel(inp, emb_0, emb_1, emb_2, emb_3, proj_0, proj_1, proj_2, proj_3):
    e2c = emb_2.reshape(_C2_CHUNK, 128)
    e3c = jnp.pad(emb_3, ((0, 224), (0, 0))).reshape(_C3_CHUNK, 128)
    w2 = _band_w(proj_2, 8, _C2_G)
    w3 = _band_w(proj_3, 2, _C3_G)
    p0s = proj_0 * EMB_SCALE
    p1s = proj_1 * EMB_SCALE

    P = _project01(emb_0, emb_1, p0s, p1s)
    P = _project_packed(P, e2c, w2, _C2_CHUNK, _C2_BASE // _C2_CHUNK, _C2_G)
    P = _project_packed(P, e3c, w3, _C3_CHUNK, _C3_BASE // _C3_CHUNK, _C3_G)

    idx = _pi(inp.astype(jnp.int32).reshape(_IDX_ROWS, 128))
    out = _gather(P, idx)
    return out.reshape(inp.shape + (D_PROJ,))
